# Initial kernel scaffold; baseline (speedup 1.0000x reference)
import jax
import jax.numpy as jnp
from jax.experimental import pallas as pl


def kernel(edge_index, emb):
    raise NotImplementedError("write your pallas kernel here")


# speedup vs baseline: 15.6082x; 15.6082x over previous
import jax
import jax.numpy as jnp
from jax import lax
from jax.experimental import pallas as pl
from jax.experimental.pallas import tpu as pltpu
from jax.experimental.pallas import tpu_sc as plsc

N_USERS = 25000
NUM_NODES = 50000
EMB_DIM = 64
H = 32
NUM_LAYERS = 3
N_EDGES = 800000

NT = 3136
NPAD = 16 * NT
TRASH = NUM_NODES
CH = 49
E_PAD = 16 * CH * 1024
E_ROWS = E_PAD // 128
NB = 112


def _body(edges, embr, out_hbm, xp_hbm,
          ridx_v, cidx_v, rows_v, dis_v, acc_b, out_b, xp_b, ones_v,
          acc_sh, deg_sh, gsem0, gsem1):
    cid = lax.axis_index("c")
    sid = lax.axis_index("s")
    z16 = jnp.zeros((16,), jnp.float32)
    one16 = jnp.full((16,), 1.0, jnp.float32)
    gsems = (gsem0, gsem1)

    nbase = sid * NT

    for j in range(8):
        ones_v[pl.ds(j * 16, 16)] = one16

    def f1(i, _):
        dis_v[pl.ds(i * 16, 16)] = z16
        return 0
    lax.fori_loop(0, NT // 16, f1, 0)

    def f2(r, _):
        xp_b[r, pl.ds(0, 16)] = z16
        xp_b[r, pl.ds(16, 16)] = z16
        return 0
    lax.fori_loop(0, NB, f2, 0)

    pltpu.sync_copy(dis_v, deg_sh.at[pl.ds(nbase, NT)])

    def z_chunk(j, _):
        pltpu.sync_copy(xp_b, acc_sh.at[pl.ds(nbase + j * NB, NB)])
        return 0
    lax.fori_loop(0, NT // NB, z_chunk, 0)
    plsc.subcore_barrier()

    def deg_chunk(c, _):
        base = (sid * CH + c) * 8
        pltpu.sync_copy(edges.at[1, pl.ds(base, 8)], cidx_v)
        for j in range(8):
            pltpu.sync_copy(ones_v, deg_sh.at[cidx_v.at[j]], add=True)
        return 0
    lax.fori_loop(0, CH, deg_chunk, 0)
    plsc.subcore_barrier()

    pltpu.sync_copy(deg_sh.at[pl.ds(nbase, NT)], dis_v)

    def disv(i, _):
        d = dis_v[pl.ds(i * 16, 16)]
        ii = lax.bitcast_convert_type(d, jnp.int32)
        ii = jnp.int32(0x5F3759DF) - (ii >> 1)
        y = lax.bitcast_convert_type(ii, jnp.float32)
        y = y * (1.5 - 0.5 * d * y * y)
        y = y * (1.5 - 0.5 * d * y * y)
        y = y * (1.5 - 0.5 * d * y * y)
        dis_v[pl.ds(i * 16, 16)] = jnp.where(d >= 0.5, y, z16)
        return 0
    lax.fori_loop(0, NT // 16, disv, 0)

    hbase = cid * NPAD + nbase

    def pre_chunk(j, _):
        pltpu.sync_copy(embr.at[pl.ds(nbase + j * NB, NB), cid], acc_b)

        def pregrp(g, _):
            dvec = dis_v[pl.ds(j * NB + g * 16, 16)]
            for i in range(16):
                r = g * 16 + i
                s = dvec[i]
                e0 = acc_b[r, pl.ds(0, 16)]
                e1 = acc_b[r, pl.ds(16, 16)]
                out_b[r, pl.ds(0, 16)] = 0.5 * e0
                out_b[r, pl.ds(16, 16)] = 0.5 * e1
                xp_b[r, pl.ds(0, 16)] = s * e0
                xp_b[r, pl.ds(16, 16)] = s * e1
            return 0
        lax.fori_loop(0, NB // 16, pregrp, 0)
        pltpu.sync_copy(out_b, out_hbm.at[pl.ds(nbase + j * NB, NB), cid])
        pltpu.sync_copy(xp_b, xp_hbm.at[pl.ds(hbase + j * NB, NB)])
        return 0
    lax.fori_loop(0, NT // NB, pre_chunk, 0)
    plsc.subcore_barrier()

    roff = cid * NPAD
    for _layer in range(NUM_LAYERS):
        def edge_chunk(c, _):
            base = (sid * CH + c) * 8
            pltpu.sync_copy(edges.at[0, pl.ds(base, 8)], ridx_v)
            pltpu.sync_copy(edges.at[1, pl.ds(base, 8)], cidx_v)
            for j in range(8):
                for v in range(8):
                    ridx_v[j, pl.ds(v * 16, 16)] = (
                        ridx_v[j, pl.ds(v * 16, 16)] + roff)
            descs = [None] * 8
            descs[0] = pltpu.async_copy(xp_hbm.at[ridx_v.at[0]],
                                        rows_v.at[0], gsems[0])
            for j in range(1, 8):
                descs[j] = pltpu.async_copy(xp_hbm.at[ridx_v.at[j]],
                                            rows_v.at[j % 2], gsems[j % 2])
                descs[j - 1].wait()
                pltpu.sync_copy(rows_v.at[(j - 1) % 2],
                                acc_sh.at[cidx_v.at[j - 1]], add=True)
            descs[7].wait()
            pltpu.sync_copy(rows_v.at[1], acc_sh.at[cidx_v.at[7]], add=True)
            return 0
        lax.fori_loop(0, CH, edge_chunk, 0)
        plsc.subcore_barrier()

        def drain_chunk(j, _):
            n0 = nbase + j * NB
            h0 = roff + n0
            pltpu.sync_copy(acc_sh.at[pl.ds(n0, NB)], acc_b)

            def fz(r, _):
                out_b[r, pl.ds(0, 16)] = z16
                out_b[r, pl.ds(16, 16)] = z16
                return 0
            lax.fori_loop(0, NB, fz, 0)
            pltpu.sync_copy(out_b, acc_sh.at[pl.ds(n0, NB)])

            pltpu.sync_copy(out_hbm.at[pl.ds(n0, NB), cid], out_b)

            def dgrp(g, _):
                dvec = dis_v[pl.ds(j * NB + g * 16, 16)]
                for i in range(16):
                    r = g * 16 + i
                    s = dvec[i]
                    a0 = acc_b[r, pl.ds(0, 16)]
                    a1 = acc_b[r, pl.ds(16, 16)]
                    x0 = s * a0
                    x1 = s * a1
                    out_b[r, pl.ds(0, 16)] = (out_b[r, pl.ds(0, 16)]
                                              + 0.5 * x0)
                    out_b[r, pl.ds(16, 16)] = (out_b[r, pl.ds(16, 16)]
                                               + 0.5 * x1)
                    xp_b[r, pl.ds(0, 16)] = s * x0
                    xp_b[r, pl.ds(16, 16)] = s * x1
                return 0
            lax.fori_loop(0, NB // 16, dgrp, 0)
            pltpu.sync_copy(out_b, out_hbm.at[pl.ds(n0, NB), cid])
            pltpu.sync_copy(xp_b, xp_hbm.at[pl.ds(h0, NB)])
            return 0
        lax.fori_loop(0, NT // NB, drain_chunk, 0)
        plsc.subcore_barrier()


_sc_prop = pl.kernel(
    _body,
    out_type=(jax.ShapeDtypeStruct((NPAD, 2, H), jnp.float32),
              jax.ShapeDtypeStruct((2 * NPAD, H), jnp.float32)),
    mesh=plsc.VectorSubcoreMesh(core_axis_name="c", subcore_axis_name="s"),
    compiler_params=pltpu.CompilerParams(use_tc_tiling_on_sc=False),
    scratch_types=[
        pltpu.VMEM((8, 128), jnp.int32),
        pltpu.VMEM((8, 128), jnp.int32),
        pltpu.VMEM((2, 128, H), jnp.float32),
        pltpu.VMEM((NT,), jnp.float32),
        pltpu.VMEM((NB, H), jnp.float32),
        pltpu.VMEM((NB, H), jnp.float32),
        pltpu.VMEM((NB, H), jnp.float32),
        pltpu.VMEM((128,), jnp.float32),
        pltpu.VMEM_SHARED((NPAD, H), jnp.float32),
        pltpu.VMEM_SHARED((NPAD,), jnp.float32),
        pltpu.SemaphoreType.DMA,
        pltpu.SemaphoreType.DMA,
    ],
)


def kernel(edge_index, emb):
    ei = edge_index.astype(jnp.int32)
    pad_n = E_PAD - N_EDGES
    pad = jnp.concatenate(
        [jnp.zeros((1, pad_n), jnp.int32),
         jnp.full((1, pad_n), TRASH, jnp.int32)], axis=0)
    ei = jnp.concatenate([ei, pad], axis=1).reshape(2, E_ROWS, 128)
    embp = jnp.pad(emb.astype(jnp.float32), ((0, NPAD - NUM_NODES), (0, 0)))
    embr = embp.reshape(NPAD, 2, H)
    out, _ = _sc_prop(ei, embr)
    o = out.reshape(NPAD, EMB_DIM)[:NUM_NODES]
    return (o[:N_USERS], o[N_USERS:])


# speedup vs baseline: 18.0532x; 1.1566x over previous
import jax
import jax.numpy as jnp
from jax import lax
from jax.experimental import pallas as pl
from jax.experimental.pallas import tpu as pltpu
from jax.experimental.pallas import tpu_sc as plsc

N_USERS = 25000
NUM_NODES = 50000
EMB_DIM = 64
H = 32
NUM_LAYERS = 3
N_EDGES = 800000

NT = 3136
NPAD = 16 * NT
E_ROWS = N_EDGES // 128
FULL_CK = E_ROWS // 8
NB = 64


def _body(edges, embr, out_hbm, xp_hbm,
          ridx_v, cidx_v, rows_v, dis_v, acc_b, out_b, ones_v,
          acc_sh, deg_sh, gs0, gs1, gs2, gs3, ss0, ss1, ss2, ss3):
    cid = lax.axis_index("c")
    sid = lax.axis_index("s")
    z16 = jnp.zeros((16,), jnp.float32)
    one16 = jnp.full((16,), 1.0, jnp.float32)
    gs = (gs0, gs1, gs2, gs3)
    ss = (ss0, ss1, ss2, ss3)

    nbase = sid * NT
    roff = cid * NPAD

    for j in range(8):
        ones_v[pl.ds(j * 16, 16)] = one16

    def f1(i, _):
        dis_v[pl.ds(i * 16, 16)] = z16
        return 0
    lax.fori_loop(0, NT // 16, f1, 0)

    def fz(r, _):
        out_b[r, pl.ds(0, 16)] = z16
        out_b[r, pl.ds(16, 16)] = z16
        return 0
    lax.fori_loop(0, NB, fz, 0)

    pltpu.sync_copy(dis_v, deg_sh.at[pl.ds(nbase, NT)])

    def z_chunk(j, _):
        pltpu.sync_copy(out_b, acc_sh.at[pl.ds(nbase + j * NB, NB)])
        return 0
    lax.fori_loop(0, NT // NB, z_chunk, 0)
    plsc.subcore_barrier()

    def deg_chunk(c, _):
        ck = c * 16 + sid

        @pl.when(ck < FULL_CK)
        def _():
            pltpu.sync_copy(edges.at[1, pl.ds(ck * 8, 8)], cidx_v)
            for j in range(8):
                pltpu.sync_copy(ones_v, deg_sh.at[cidx_v.at[j]], add=True)
        return 0
    lax.fori_loop(0, 49, deg_chunk, 0)

    @pl.when(sid == 15)
    def _():
        pltpu.sync_copy(edges.at[1, pl.ds(FULL_CK * 8, 2)],
                        cidx_v.at[pl.ds(0, 2)])
        for j in range(2):
            pltpu.sync_copy(ones_v, deg_sh.at[cidx_v.at[j]], add=True)
    plsc.subcore_barrier()

    pltpu.sync_copy(deg_sh.at[pl.ds(nbase, NT)], dis_v)

    def disv(i, _):
        d = dis_v[pl.ds(i * 16, 16)]
        ii = lax.bitcast_convert_type(d, jnp.int32)
        ii = jnp.int32(0x5F3759DF) - (ii >> 1)
        y = lax.bitcast_convert_type(ii, jnp.float32)
        y = y * (1.5 - 0.5 * d * y * y)
        y = y * (1.5 - 0.5 * d * y * y)
        y = y * (1.5 - 0.5 * d * y * y)
        dis_v[pl.ds(i * 16, 16)] = jnp.where(d >= 0.5, y, z16)
        return 0
    lax.fori_loop(0, NT // 16, disv, 0)

    def _prescale(j, nrows):
        node0 = nbase + j * NB
        pltpu.sync_copy(embr.at[pl.ds(node0, nrows), cid],
                        acc_b.at[pl.ds(0, nrows)])

        def grp(g, _):
            dvec = dis_v[pl.ds(j * NB + g * 16, 16)]
            for i in range(16):
                r = g * 16 + i
                s = dvec[i]
                e0 = acc_b[r, pl.ds(0, 16)]
                e1 = acc_b[r, pl.ds(16, 16)]
                out_b[r, pl.ds(0, 16)] = 0.5 * e0
                out_b[r, pl.ds(16, 16)] = 0.5 * e1
                acc_b[r, pl.ds(0, 16)] = s * e0
                acc_b[r, pl.ds(16, 16)] = s * e1
            return 0
        lax.fori_loop(0, nrows // 16, grp, 0)
        pltpu.sync_copy(out_b.at[pl.ds(0, nrows)],
                        out_hbm.at[pl.ds(node0, nrows), cid])
        pltpu.sync_copy(acc_b.at[pl.ds(0, nrows)],
                        xp_hbm.at[pl.ds(roff + node0, nrows)])

    def pre_chunk(j, _):
        node0 = nbase + j * NB

        @pl.when(node0 + NB <= NUM_NODES)
        def _():
            _prescale(j, NB)

        @pl.when(jnp.logical_and(node0 < NUM_NODES,
                                 node0 + NB > NUM_NODES))
        def _():
            _prescale(j, 16)
        return 0
    lax.fori_loop(0, NT // NB, pre_chunk, 0)
    plsc.subcore_barrier()

    for _layer in range(NUM_LAYERS):
        def edge_chunk(c, _):
            ck = c * 16 + sid

            @pl.when(ck < FULL_CK)
            def _():
                base = ck * 8
                pltpu.sync_copy(edges.at[0, pl.ds(base, 8)], ridx_v)
                pltpu.sync_copy(edges.at[1, pl.ds(base, 8)], cidx_v)
                for j in range(8):
                    for v in range(8):
                        ridx_v[j, pl.ds(v * 16, 16)] = (
                            ridx_v[j, pl.ds(v * 16, 16)] + roff)
                g = [pltpu.async_copy(xp_hbm.at[ridx_v.at[j]],
                                      rows_v.at[j], gs[j])
                     for j in range(4)]
                s = [None] * 8
                for j in range(4):
                    g[j].wait()
                    s[j] = pltpu.async_copy(rows_v.at[j],
                                            acc_sh.at[cidx_v.at[j]],
                                            ss[j], add=True)
                g2 = [None] * 4
                for j in range(4):
                    s[j].wait()
                    g2[j] = pltpu.async_copy(xp_hbm.at[ridx_v.at[4 + j]],
                                             rows_v.at[j], gs[j])
                for j in range(4):
                    g2[j].wait()
                    s[4 + j] = pltpu.async_copy(rows_v.at[j],
                                                acc_sh.at[cidx_v.at[4 + j]],
                                                ss[j], add=True)
                for j in range(4):
                    s[4 + j].wait()
            return 0
        lax.fori_loop(0, 49, edge_chunk, 0)

        @pl.when(sid == 15)
        def _():
            pltpu.sync_copy(edges.at[0, pl.ds(FULL_CK * 8, 2)],
                            ridx_v.at[pl.ds(0, 2)])
            pltpu.sync_copy(edges.at[1, pl.ds(FULL_CK * 8, 2)],
                            cidx_v.at[pl.ds(0, 2)])
            for j in range(2):
                for v in range(8):
                    ridx_v[j, pl.ds(v * 16, 16)] = (
                        ridx_v[j, pl.ds(v * 16, 16)] + roff)
            for j in range(2):
                pltpu.async_copy(xp_hbm.at[ridx_v.at[j]], rows_v.at[j],
                                 gs[j]).wait()
                pltpu.sync_copy(rows_v.at[j], acc_sh.at[cidx_v.at[j]],
                                add=True)
        plsc.subcore_barrier()

        def _drain(j, nrows):
            node0 = nbase + j * NB
            pltpu.sync_copy(acc_sh.at[pl.ds(node0, nrows)],
                            acc_b.at[pl.ds(0, nrows)])
            pltpu.sync_copy(out_hbm.at[pl.ds(node0, nrows), cid],
                            out_b.at[pl.ds(0, nrows)])

            def grp(g, _):
                dvec = dis_v[pl.ds(j * NB + g * 16, 16)]
                for i in range(16):
                    r = g * 16 + i
                    s = dvec[i]
                    a0 = acc_b[r, pl.ds(0, 16)]
                    a1 = acc_b[r, pl.ds(16, 16)]
                    x0 = s * a0
                    x1 = s * a1
                    out_b[r, pl.ds(0, 16)] = (out_b[r, pl.ds(0, 16)]
                                              + 0.5 * x0)
                    out_b[r, pl.ds(16, 16)] = (out_b[r, pl.ds(16, 16)]
                                               + 0.5 * x1)
                    acc_b[r, pl.ds(0, 16)] = s * x0
                    acc_b[r, pl.ds(16, 16)] = s * x1
                return 0
            lax.fori_loop(0, nrows // 16, grp, 0)
            pltpu.sync_copy(out_b.at[pl.ds(0, nrows)],
                            out_hbm.at[pl.ds(node0, nrows), cid])
            pltpu.sync_copy(acc_b.at[pl.ds(0, nrows)],
                            xp_hbm.at[pl.ds(roff + node0, nrows)])
            def fz2(r, _):
                out_b[r, pl.ds(0, 16)] = z16
                out_b[r, pl.ds(16, 16)] = z16
                return 0
            lax.fori_loop(0, nrows, fz2, 0)
            pltpu.sync_copy(out_b.at[pl.ds(0, nrows)],
                            acc_sh.at[pl.ds(node0, nrows)])

        def drain_chunk(j, _):
            node0 = nbase + j * NB

            @pl.when(node0 + NB <= NUM_NODES)
            def _():
                _drain(j, NB)

            @pl.when(jnp.logical_and(node0 < NUM_NODES,
                                     node0 + NB > NUM_NODES))
            def _():
                _drain(j, 16)
            return 0
        lax.fori_loop(0, NT // NB, drain_chunk, 0)
        plsc.subcore_barrier()


_sc_prop = pl.kernel(
    _body,
    out_type=(jax.ShapeDtypeStruct((NUM_NODES, 2, H), jnp.float32),
              jax.ShapeDtypeStruct((2 * NPAD, H), jnp.float32)),
    mesh=plsc.VectorSubcoreMesh(core_axis_name="c", subcore_axis_name="s"),
    compiler_params=pltpu.CompilerParams(use_tc_tiling_on_sc=False),
    scratch_types=[
        pltpu.VMEM((8, 128), jnp.int32),
        pltpu.VMEM((8, 128), jnp.int32),
        pltpu.VMEM((4, 128, H), jnp.float32),
        pltpu.VMEM((NT,), jnp.float32),
        pltpu.VMEM((NB, H), jnp.float32),
        pltpu.VMEM((NB, H), jnp.float32),
        pltpu.VMEM((128,), jnp.float32),
        pltpu.VMEM_SHARED((NPAD, H), jnp.float32),
        pltpu.VMEM_SHARED((NPAD,), jnp.float32),
        pltpu.SemaphoreType.DMA,
        pltpu.SemaphoreType.DMA,
        pltpu.SemaphoreType.DMA,
        pltpu.SemaphoreType.DMA,
        pltpu.SemaphoreType.DMA,
        pltpu.SemaphoreType.DMA,
        pltpu.SemaphoreType.DMA,
        pltpu.SemaphoreType.DMA,
    ],
)


def kernel(edge_index, emb):
    ei = edge_index.astype(jnp.int32).reshape(2, E_ROWS, 128)
    embr = emb.astype(jnp.float32).reshape(NUM_NODES, 2, H)
    out, _ = _sc_prop(ei, embr)
    o = out.reshape(NUM_NODES, EMB_DIM)
    return (o[:N_USERS], o[N_USERS:])


# speedup vs baseline: 21.5131x; 1.1917x over previous
import jax
import jax.numpy as jnp
from jax import lax
from jax.experimental import pallas as pl
from jax.experimental.pallas import tpu as pltpu
from jax.experimental.pallas import tpu_sc as plsc

N_USERS = 25000
NUM_NODES = 50000
EMB_DIM = 64
H = 32
NUM_LAYERS = 3
N_EDGES = 800000

NT = 3136
NPAD = 16 * NT
FULL_CK = N_EDGES // 1024
TAIL0 = FULL_CK * 1024
NB = 64


def _body(edges, embr, usr_hbm, itm_hbm, xp_hbm,
          ridx_v, cstg_v, cidx_v, rows_v, dis_v, acc_b, out_b, ones_v,
          acc_sh, deg_sh, gs0, gs1, gs2, gs3, ss0, ss1, ss2, ss3):
    cid = lax.axis_index("c")
    sid = lax.axis_index("s")
    z16 = jnp.zeros((16,), jnp.float32)
    one16 = jnp.full((16,), 1.0, jnp.float32)
    gs = (gs0, gs1, gs2, gs3)
    ss = (ss0, ss1, ss2, ss3)

    nbase = sid * NT
    roff = cid * NPAD
    fbase = cid * H

    for j in range(8):
        ones_v[pl.ds(j * 16, 16)] = one16

    def f1(i, _):
        dis_v[pl.ds(i * 16, 16)] = z16
        return 0
    lax.fori_loop(0, NT // 16, f1, 0)

    def fz(r, _):
        out_b[r, pl.ds(0, 16)] = z16
        out_b[r, pl.ds(16, 16)] = z16
        return 0
    lax.fori_loop(0, NB, fz, 0)

    pltpu.sync_copy(dis_v, deg_sh.at[pl.ds(nbase, NT)])

    def z_chunk(j, _):
        pltpu.sync_copy(out_b, acc_sh.at[pl.ds(nbase + j * NB, NB)])
        return 0
    lax.fori_loop(0, NT // NB, z_chunk, 0)
    plsc.subcore_barrier()

    def load_cidx(base_e, nblk):
        pltpu.sync_copy(edges.at[1, pl.ds(base_e, nblk * 128)],
                        cstg_v.at[pl.ds(0, nblk * 128)])
        for j in range(nblk):
            for v in range(8):
                cidx_v[j, pl.ds(v * 16, 16)] = (
                    cstg_v[pl.ds((j * 8 + v) * 16, 16)])

    def load_ridx(base_e, nblk):
        pltpu.sync_copy(edges.at[0, pl.ds(base_e, nblk * 128)],
                        ridx_v.at[pl.ds(0, nblk * 128)])
        for v in range(nblk * 8):
            ridx_v[pl.ds(v * 16, 16)] = ridx_v[pl.ds(v * 16, 16)] + roff

    def deg_chunk(c, _):
        ck = c * 16 + sid

        @pl.when(ck < FULL_CK)
        def _():
            load_cidx(ck * 1024, 8)
            for j in range(8):
                pltpu.sync_copy(ones_v, deg_sh.at[cidx_v.at[j]], add=True)
        return 0
    lax.fori_loop(0, 49, deg_chunk, 0)

    @pl.when(sid == 15)
    def _():
        load_cidx(TAIL0, 2)
        for j in range(2):
            pltpu.sync_copy(ones_v, deg_sh.at[cidx_v.at[j]], add=True)
    plsc.subcore_barrier()

    pltpu.sync_copy(deg_sh.at[pl.ds(nbase, NT)], dis_v)

    def disv(i, _):
        d = dis_v[pl.ds(i * 16, 16)]
        ii = lax.bitcast_convert_type(d, jnp.int32)
        ii = jnp.int32(0x5F3759DF) - (ii >> 1)
        y = lax.bitcast_convert_type(ii, jnp.float32)
        y = y * (1.5 - 0.5 * d * y * y)
        y = y * (1.5 - 0.5 * d * y * y)
        y = y * (1.5 - 0.5 * d * y * y)
        dis_v[pl.ds(i * 16, 16)] = jnp.where(d >= 0.5, y, z16)
        return 0
    lax.fori_loop(0, NT // 16, disv, 0)

    def out_copy(node0, nrows, to_hbm):
        def cp(hbm, r0, bb, nn):
            if to_hbm:
                pltpu.sync_copy(out_b.at[pl.ds(bb, nn)],
                                hbm.at[pl.ds(r0, nn), pl.ds(fbase, H)])
            else:
                pltpu.sync_copy(hbm.at[pl.ds(r0, nn), pl.ds(fbase, H)],
                                out_b.at[pl.ds(bb, nn)])

        @pl.when(node0 + nrows <= N_USERS)
        def _():
            cp(usr_hbm, node0, 0, nrows)

        @pl.when(node0 >= N_USERS)
        def _():
            cp(itm_hbm, node0 - N_USERS, 0, nrows)

        if nrows == NB:
            @pl.when(node0 == N_USERS - 40)
            def _():
                cp(usr_hbm, N_USERS - 40, 0, 40)
                cp(itm_hbm, 0, 40, NB - 40)

    def _prescale(j, nrows):
        node0 = nbase + j * NB
        pltpu.sync_copy(embr.at[pl.ds(node0, nrows), pl.ds(fbase, H)],
                        acc_b.at[pl.ds(0, nrows)])

        def grp(g, _):
            dvec = dis_v[pl.ds(j * NB + g * 16, 16)]
            for i in range(16):
                r = g * 16 + i
                s = dvec[i]
                e0 = acc_b[r, pl.ds(0, 16)]
                e1 = acc_b[r, pl.ds(16, 16)]
                out_b[r, pl.ds(0, 16)] = 0.5 * e0
                out_b[r, pl.ds(16, 16)] = 0.5 * e1
                acc_b[r, pl.ds(0, 16)] = s * e0
                acc_b[r, pl.ds(16, 16)] = s * e1
            return 0
        lax.fori_loop(0, nrows // 16, grp, 0)
        out_copy(node0, nrows, True)
        pltpu.sync_copy(acc_b.at[pl.ds(0, nrows)],
                        xp_hbm.at[pl.ds(roff + node0, nrows)])

    def pre_chunk(j, _):
        node0 = nbase + j * NB

        @pl.when(node0 + NB <= NUM_NODES)
        def _():
            _prescale(j, NB)

        @pl.when(jnp.logical_and(node0 < NUM_NODES,
                                 node0 + NB > NUM_NODES))
        def _():
            _prescale(j, 16)
        return 0
    lax.fori_loop(0, NT // NB, pre_chunk, 0)
    plsc.subcore_barrier()

    for _layer in range(NUM_LAYERS):
        def edge_chunk(c, _):
            ck = c * 16 + sid

            @pl.when(ck < FULL_CK)
            def _():
                base_e = ck * 1024
                load_ridx(base_e, 8)
                load_cidx(base_e, 8)
                g = [pltpu.async_copy(
                        xp_hbm.at[ridx_v.at[pl.ds(j * 128, 128)]],
                        rows_v.at[j], gs[j])
                     for j in range(4)]
                s = [None] * 8
                for j in range(4):
                    g[j].wait()
                    s[j] = pltpu.async_copy(rows_v.at[j],
                                            acc_sh.at[cidx_v.at[j]],
                                            ss[j], add=True)
                g2 = [None] * 4
                for j in range(4):
                    s[j].wait()
                    g2[j] = pltpu.async_copy(
                        xp_hbm.at[ridx_v.at[pl.ds((4 + j) * 128, 128)]],
                        rows_v.at[j], gs[j])
                for j in range(4):
                    g2[j].wait()
                    s[4 + j] = pltpu.async_copy(rows_v.at[j],
                                                acc_sh.at[cidx_v.at[4 + j]],
                                                ss[j], add=True)
                for j in range(4):
                    s[4 + j].wait()
            return 0
        lax.fori_loop(0, 49, edge_chunk, 0)

        @pl.when(sid == 15)
        def _():
            load_ridx(TAIL0, 2)
            load_cidx(TAIL0, 2)
            for j in range(2):
                pltpu.async_copy(
                    xp_hbm.at[ridx_v.at[pl.ds(j * 128, 128)]],
                    rows_v.at[j], gs[j]).wait()
                pltpu.sync_copy(rows_v.at[j], acc_sh.at[cidx_v.at[j]],
                                add=True)
        plsc.subcore_barrier()

        def _drain(j, nrows):
            node0 = nbase + j * NB
            pltpu.sync_copy(acc_sh.at[pl.ds(node0, nrows)],
                            acc_b.at[pl.ds(0, nrows)])
            out_copy(node0, nrows, False)

            def grp(g, _):
                dvec = dis_v[pl.ds(j * NB + g * 16, 16)]
                for i in range(16):
                    r = g * 16 + i
                    s = dvec[i]
                    a0 = acc_b[r, pl.ds(0, 16)]
                    a1 = acc_b[r, pl.ds(16, 16)]
                    x0 = s * a0
                    x1 = s * a1
                    out_b[r, pl.ds(0, 16)] = (out_b[r, pl.ds(0, 16)]
                                              + 0.5 * x0)
                    out_b[r, pl.ds(16, 16)] = (out_b[r, pl.ds(16, 16)]
                                               + 0.5 * x1)
                    acc_b[r, pl.ds(0, 16)] = s * x0
                    acc_b[r, pl.ds(16, 16)] = s * x1
                return 0
            lax.fori_loop(0, nrows // 16, grp, 0)
            out_copy(node0, nrows, True)
            pltpu.sync_copy(acc_b.at[pl.ds(0, nrows)],
                            xp_hbm.at[pl.ds(roff + node0, nrows)])
            def fz2(r, _):
                out_b[r, pl.ds(0, 16)] = z16
                out_b[r, pl.ds(16, 16)] = z16
                return 0
            lax.fori_loop(0, nrows, fz2, 0)
            pltpu.sync_copy(out_b.at[pl.ds(0, nrows)],
                            acc_sh.at[pl.ds(node0, nrows)])

        def drain_chunk(j, _):
            node0 = nbase + j * NB

            @pl.when(node0 + NB <= NUM_NODES)
            def _():
                _drain(j, NB)

            @pl.when(jnp.logical_and(node0 < NUM_NODES,
                                     node0 + NB > NUM_NODES))
            def _():
                _drain(j, 16)
            return 0
        lax.fori_loop(0, NT // NB, drain_chunk, 0)
        plsc.subcore_barrier()


_sc_prop = pl.kernel(
    _body,
    out_type=(jax.ShapeDtypeStruct((N_USERS, EMB_DIM), jnp.float32),
              jax.ShapeDtypeStruct((NUM_NODES - N_USERS, EMB_DIM),
                                   jnp.float32),
              jax.ShapeDtypeStruct((2 * NPAD, H), jnp.float32)),
    mesh=plsc.VectorSubcoreMesh(core_axis_name="c", subcore_axis_name="s"),
    compiler_params=pltpu.CompilerParams(use_tc_tiling_on_sc=False),
    scratch_types=[
        pltpu.VMEM((1024,), jnp.int32),
        pltpu.VMEM((1024,), jnp.int32),
        pltpu.VMEM((8, 128), jnp.int32),
        pltpu.VMEM((4, 128, H), jnp.float32),
        pltpu.VMEM((NT,), jnp.float32),
        pltpu.VMEM((NB, H), jnp.float32),
        pltpu.VMEM((NB, H), jnp.float32),
        pltpu.VMEM((128,), jnp.float32),
        pltpu.VMEM_SHARED((NPAD, H), jnp.float32),
        pltpu.VMEM_SHARED((NPAD,), jnp.float32),
        pltpu.SemaphoreType.DMA,
        pltpu.SemaphoreType.DMA,
        pltpu.SemaphoreType.DMA,
        pltpu.SemaphoreType.DMA,
        pltpu.SemaphoreType.DMA,
        pltpu.SemaphoreType.DMA,
        pltpu.SemaphoreType.DMA,
        pltpu.SemaphoreType.DMA,
    ],
)


def kernel(edge_index, emb):
    usr, itm, _ = _sc_prop(edge_index.astype(jnp.int32),
                           emb.astype(jnp.float32))
    return (usr, itm)
